# baseline (speedup 1.0000x reference)
import jax
import jax.numpy as jnp
import numpy as np
from jax.experimental import pallas as pl
from jax.experimental.pallas import tpu as pltpu

_B, _NC, _NT = 16, 512, 128
_K = 8
_H = 64
_NBLK = 6
_N = _NC + _NT
_BPP = 16

_f32 = jnp.float32


def _ln(x, g, b):
    mu = jnp.mean(x, axis=-1, keepdims=True)
    var = jnp.mean((x - mu) ** 2, axis=-1, keepdims=True)
    return (x - mu) * jax.lax.rsqrt(var + 1e-6) * g + b


def _dot(a, b):
    return jnp.dot(a, b, preferred_element_type=_f32)


def _sgnp_kernel(
    s_all_ref,
    f_ref,
    tmask_ref,
    w1_ref, bc_ref,
    w2_ref, b2_ref, w3_ref, b3_ref,
    ng_ref, nb_ref,
    wall_ref,
    we_ref, wo_ref,
    wf1_ref, bf1_ref, wf2_ref, bf2_ref,
    l1g_ref, l1b_ref, l2g_ref, l2b_ref,
    hw1_ref, hb1_ref, hw2_ref, hb2_ref, hw3_ref, hb3_ref,
    out_ref,
):
    coords = s_all_ref[...].reshape(_BPP * _N, 2)
    fpad = f_ref[...].reshape(_BPP * _N, 1)

    x4 = jnp.concatenate([coords, fpad, tmask_ref[...]], axis=1)
    h1 = _dot(x4, w1_ref[...]) + bc_ref[...]
    h2 = jax.nn.gelu(_dot(jax.nn.gelu(h1), w2_ref[...]) + b2_ref[...])
    h = _ln(_dot(h2, w3_ref[...]) + b3_ref[...], ng_ref[...], nb_ref[...])

    lane = jax.lax.broadcasted_iota(jnp.int32, (_N, _NC), 1)
    masks = []
    s2cs = []
    qcols = []
    for i in range(_BPP):
        qxy = coords[i * _N:(i + 1) * _N]
        s2c = qxy[:_NC]
        qx = qxy[:, 0:1]
        qy = qxy[:, 1:2]
        ddx = qx - jnp.transpose(s2c[:, 0:1])
        ddy = qy - jnp.transpose(s2c[:, 1:2])
        cur = ddx * ddx + ddy * ddy
        for _ in range(_K):
            rmin = jnp.min(cur, axis=1, keepdims=True)
            cand = jnp.where(cur == rmin, lane, jnp.int32(1 << 20))
            jmin = jnp.min(cand, axis=1, keepdims=True)
            cur = jnp.where(lane == jmin, jnp.inf, cur)
        masks.append(cur == jnp.inf)
        s2cs.append(s2c)
        qcols.append((qx, qy))

    for blk in range(_NBLK):
        allm = _dot(h, wall_ref[blk])
        wex = we_ref[blk, 0:1, :]
        wey = we_ref[blk, 1:2, :]
        msgs = []
        for i in range(_BPP):
            base = i * _N
            qs = allm[base:base + _N, 0:66]
            kc = allm[base:base + _NC, 128:192]
            vc = allm[base:base + _NC, 192:256]
            k_aug = jnp.concatenate([kc, s2cs[i]], axis=1)
            v_aug = jnp.concatenate([vc, s2cs[i]], axis=1)
            logits = jax.lax.dot_general(
                qs, k_aug, (((1,), (1,)), ((), ())),
                preferred_element_type=_f32)
            lm = jnp.where(masks[i], logits, -1e30)
            mx = jnp.max(lm, axis=1, keepdims=True)
            ex = jnp.exp(lm - mx)
            den = jnp.sum(ex, axis=1, keepdims=True)
            inv = 1.0 / (den + 1e-9)
            agg = _dot(ex, v_aug) * inv
            rs = den * inv
            qx, qy = qcols[i]
            msgs.append(agg[:, :64]
                        + (agg[:, 64:65] - rs * qx) * wex
                        + (agg[:, 65:66] - rs * qy) * wey)
        msg = jnp.concatenate(msgs, axis=0)
        h = _ln(h + _dot(msg, wo_ref[blk]), l1g_ref[blk], l1b_ref[blk])
        ff = _dot(jax.nn.gelu(_dot(h, wf1_ref[blk]) + bf1_ref[blk]),
                  wf2_ref[blk]) + bf2_ref[blk]
        h = _ln(h + ff, l2g_ref[blk], l2b_ref[blk])

    xt = jnp.concatenate(
        [h[i * _N + _NC:(i + 1) * _N] for i in range(_BPP)], axis=0)
    y = jax.nn.gelu(_dot(xt, hw1_ref[...]) + hb1_ref[...])
    y = jax.nn.gelu(_dot(y, hw2_ref[...]) + hb2_ref[...])
    y = _dot(y, hw3_ref[...]) + hb3_ref[...]
    out_ref[...] = jnp.concatenate(
        [y[:, 0:1], jax.nn.softplus(y[:, 1:2])], axis=1).reshape(_BPP, _NT, 2)


def _const_spec(arr):
    nd = arr.ndim
    return pl.BlockSpec(arr.shape, lambda b, _n=nd: (0,) * _n)


@jax.jit
def kernel(s_ctx, f_ctx, s_test, params):
    p = params
    s_all = jnp.concatenate([s_ctx, s_test], axis=1)
    f_pad = jnp.concatenate(
        [f_ctx, jnp.zeros((_B, _NT, 1), _f32)], axis=1)
    tmask = jnp.asarray(
        np.tile(np.repeat([0.0, 1.0], [_NC, _NT]), _BPP)[:, None], _f32)
    w1 = p['embed_all_W'][0]
    b1 = p['embed_all_b'][0]
    bc = p['embed_obs'][1:2] @ w1[:4] + b1[None]
    bt = p['embed_obs'][0:1] @ w1[:4] + b1[None]
    row = lambda v: v.reshape(1, -1)

    inv_sqrt = 1.0 / (_H ** 0.5)
    wq, wk, wv, we = p['gat_Wq'], p['gat_Wk'], p['gat_Wv'], p['gat_We']
    ucols = jnp.einsum('bij,bkj->bik', wq, we)
    wq_aug = jnp.concatenate(
        [wq, ucols, jnp.zeros((_NBLK, _H, 62), _f32)], axis=2) * inv_sqrt
    w_all = jnp.concatenate([wq_aug, wk, wv], axis=2)

    operands = [
        s_all, f_pad, tmask,
        jnp.concatenate([w1[4:7], bt - bc], axis=0), bc,
        p['embed_all_W'][1], row(p['embed_all_b'][1]),
        p['embed_all_W'][2], row(p['embed_all_b'][2]),
        row(p['norm_g']), row(p['norm_b']),
        w_all, we, p['gat_Wo'],
        p['gat_ffn_W1'], p['gat_ffn_b1'][:, None, :],
        p['gat_ffn_W2'], p['gat_ffn_b2'][:, None, :],
        p['gat_ln1_g'][:, None, :], p['gat_ln1_b'][:, None, :],
        p['gat_ln2_g'][:, None, :], p['gat_ln2_b'][:, None, :],
        p['head_W'][0], row(p['head_b'][0]),
        p['head_W'][1], row(p['head_b'][1]),
        p['head_W'][2], row(p['head_b'][2]),
    ]
    in_specs = [
        pl.BlockSpec((_BPP, _N, 2), lambda b: (b, 0, 0)),
        pl.BlockSpec((_BPP, _N, 1), lambda b: (b, 0, 0)),
    ] + [_const_spec(a) for a in operands[2:]]

    out = pl.pallas_call(
        _sgnp_kernel,
        grid=(_B // _BPP,),
        in_specs=in_specs,
        out_specs=pl.BlockSpec((_BPP, _NT, 2), lambda b: (b, 0, 0)),
        out_shape=jax.ShapeDtypeStruct((_B, _NT, 2), _f32),
        compiler_params=pltpu.CompilerParams(
            dimension_semantics=("parallel",),
            vmem_limit_bytes=100 * 1024 * 1024),
    )(*operands)
    return out


# speedup vs baseline: 1.0082x; 1.0082x over previous
import jax
import jax.numpy as jnp
import numpy as np
from jax.experimental import pallas as pl
from jax.experimental.pallas import tpu as pltpu

_B, _NC, _NT = 16, 512, 128
_K = 8
_H = 64
_NBLK = 6
_N = _NC + _NT
_BPP = 16

_f32 = jnp.float32


def _ln(x, g, b):
    mu = jnp.mean(x, axis=-1, keepdims=True)
    var = jnp.mean((x - mu) ** 2, axis=-1, keepdims=True)
    return (x - mu) * jax.lax.rsqrt(var + 1e-6) * g + b


def _dot(a, b):
    return jnp.dot(a, b, preferred_element_type=_f32)


def _sgnp_kernel(
    s_all_ref,
    f_ref,
    tmask_ref,
    w1_ref, bc_ref, dbt_ref,
    w2_ref, b2_ref, w3_ref, b3_ref,
    ng_ref, nb_ref,
    wall_ref,
    we_ref, wo_ref,
    wf1_ref, bf1_ref, wf2_ref, bf2_ref,
    l1g_ref, l1b_ref, l2g_ref, l2b_ref,
    hw1_ref, hb1_ref, hw2_ref, hb2_ref, hw3_ref, hb3_ref,
    out_ref,
):
    coords = s_all_ref[...].reshape(_BPP * _N, 2)
    fpad = f_ref[...].reshape(_BPP * _N, 1)

    x3 = jnp.concatenate([coords, fpad], axis=1)
    h1 = _dot(x3, w1_ref[...]) + bc_ref[...] + tmask_ref[...] * dbt_ref[...]
    h2 = jax.nn.gelu(_dot(jax.nn.gelu(h1), w2_ref[...]) + b2_ref[...])
    h = _ln(_dot(h2, w3_ref[...]) + b3_ref[...], ng_ref[...], nb_ref[...])

    lane = jax.lax.broadcasted_iota(jnp.int32, (_N, _NC), 1)
    masks = []
    s2cs = []
    qcols = []
    for i in range(_BPP):
        qxy = coords[i * _N:(i + 1) * _N]
        s2c = qxy[:_NC]
        qx = qxy[:, 0:1]
        qy = qxy[:, 1:2]
        ddx = qx - jnp.transpose(s2c[:, 0:1])
        ddy = qy - jnp.transpose(s2c[:, 1:2])
        cur = ddx * ddx + ddy * ddy
        for _ in range(_K):
            rmin = jnp.min(cur, axis=1, keepdims=True)
            cand = jnp.where(cur == rmin, lane, jnp.int32(1 << 20))
            jmin = jnp.min(cand, axis=1, keepdims=True)
            cur = jnp.where(lane == jmin, jnp.inf, cur)
        masks.append(cur == jnp.inf)
        s2cs.append(s2c)
        qcols.append((qx, qy))

    for blk in range(_NBLK):
        allm = _dot(h, wall_ref[blk])
        wex = we_ref[blk, 0:1, :]
        wey = we_ref[blk, 1:2, :]
        msgs = []
        for i in range(_BPP):
            base = i * _N
            qs = allm[base:base + _N, 0:66]
            kc = allm[base:base + _NC, 128:192]
            vc = allm[base:base + _NC, 192:256]
            k_aug = jnp.concatenate([kc, s2cs[i]], axis=1)
            v_aug = jnp.concatenate([vc, s2cs[i]], axis=1)
            logits = jax.lax.dot_general(
                qs, k_aug, (((1,), (1,)), ((), ())),
                preferred_element_type=_f32)
            lm = jnp.where(masks[i], logits, -1e30)
            mx = jnp.max(lm, axis=1, keepdims=True)
            ex = jnp.exp(lm - mx)
            den = jnp.sum(ex, axis=1, keepdims=True)
            inv = 1.0 / (den + 1e-9)
            agg = _dot(ex, v_aug) * inv
            rs = den * inv
            qx, qy = qcols[i]
            msgs.append(agg[:, :64]
                        + (agg[:, 64:65] - rs * qx) * wex
                        + (agg[:, 65:66] - rs * qy) * wey)
        msg = jnp.concatenate(msgs, axis=0)
        h = _ln(h + _dot(msg, wo_ref[blk]), l1g_ref[blk], l1b_ref[blk])
        ff = _dot(jax.nn.gelu(_dot(h, wf1_ref[blk]) + bf1_ref[blk]),
                  wf2_ref[blk]) + bf2_ref[blk]
        h = _ln(h + ff, l2g_ref[blk], l2b_ref[blk])

    xt = jnp.concatenate(
        [h[i * _N + _NC:(i + 1) * _N] for i in range(_BPP)], axis=0)
    y = jax.nn.gelu(_dot(xt, hw1_ref[...]) + hb1_ref[...])
    y = jax.nn.gelu(_dot(y, hw2_ref[...]) + hb2_ref[...])
    y = _dot(y, hw3_ref[...]) + hb3_ref[...]
    out_ref[...] = jnp.concatenate(
        [y[:, 0:1], jax.nn.softplus(y[:, 1:2])], axis=1).reshape(_BPP, _NT, 2)


def _const_spec(arr):
    nd = arr.ndim
    return pl.BlockSpec(arr.shape, lambda b, _n=nd: (0,) * _n)


@jax.jit
def kernel(s_ctx, f_ctx, s_test, params):
    p = params
    s_all = jnp.concatenate([s_ctx, s_test], axis=1)
    f_pad = jnp.concatenate(
        [f_ctx, jnp.zeros((_B, _NT, 1), _f32)], axis=1)
    tmask = jnp.asarray(
        np.tile(np.repeat([0.0, 1.0], [_NC, _NT]), _BPP)[:, None], _f32)
    w1 = p['embed_all_W'][0]
    b1 = p['embed_all_b'][0]
    bc = p['embed_obs'][1:2] @ w1[:4] + b1[None]
    bt = p['embed_obs'][0:1] @ w1[:4] + b1[None]
    row = lambda v: v.reshape(1, -1)

    inv_sqrt = 1.0 / (_H ** 0.5)
    wq, wk, wv, we = p['gat_Wq'], p['gat_Wk'], p['gat_Wv'], p['gat_We']
    ucols = jnp.einsum('bij,bkj->bik', wq, we)
    wq_aug = jnp.concatenate(
        [wq, ucols, jnp.zeros((_NBLK, _H, 62), _f32)], axis=2) * inv_sqrt
    w_all = jnp.concatenate([wq_aug, wk, wv], axis=2)

    operands = [
        s_all, f_pad, tmask,
        w1[4:7], bc, bt - bc,
        p['embed_all_W'][1], row(p['embed_all_b'][1]),
        p['embed_all_W'][2], row(p['embed_all_b'][2]),
        row(p['norm_g']), row(p['norm_b']),
        w_all, we, p['gat_Wo'],
        p['gat_ffn_W1'], p['gat_ffn_b1'][:, None, :],
        p['gat_ffn_W2'], p['gat_ffn_b2'][:, None, :],
        p['gat_ln1_g'][:, None, :], p['gat_ln1_b'][:, None, :],
        p['gat_ln2_g'][:, None, :], p['gat_ln2_b'][:, None, :],
        p['head_W'][0], row(p['head_b'][0]),
        p['head_W'][1], row(p['head_b'][1]),
        p['head_W'][2], row(p['head_b'][2]),
    ]
    in_specs = [
        pl.BlockSpec((_BPP, _N, 2), lambda b: (b, 0, 0)),
        pl.BlockSpec((_BPP, _N, 1), lambda b: (b, 0, 0)),
    ] + [_const_spec(a) for a in operands[2:]]

    out = pl.pallas_call(
        _sgnp_kernel,
        grid=(_B // _BPP,),
        in_specs=in_specs,
        out_specs=pl.BlockSpec((_BPP, _NT, 2), lambda b: (b, 0, 0)),
        out_shape=jax.ShapeDtypeStruct((_B, _NT, 2), _f32),
        compiler_params=pltpu.CompilerParams(
            dimension_semantics=("parallel",),
            vmem_limit_bytes=100 * 1024 * 1024),
    )(*operands)
    return out
